# baseline (speedup 1.0000x reference)
import jax
import jax.numpy as jnp
from jax import lax
from jax.experimental import pallas as pl
from jax.experimental.pallas import tpu as pltpu
from jax.experimental.pallas import tpu_sc as plsc

NC = 2
NS = 16
L = 16
NW = NC * NS

DIM = 64
BLK = 128
NBUF = 4
LOOKAHEAD = 2


def _sc_body(xt_hbm, table_hbm, out_hbm, idx_v, gbufs, obufs, gsems, osems):
    wid = lax.axis_index("s") * NC + lax.axis_index("c")
    fields = xt_hbm.shape[0]
    batches = xt_hbm.shape[1]
    blocks_per_w = batches // (BLK * NW)
    n_chunks = blocks_per_w * fields
    base_blk = wid * blocks_per_w

    pltpu.sync_copy(xt_hbm.at[:, pl.ds(base_blk * BLK, blocks_per_w * BLK)], idx_v)

    row_sel = [lax.iota(jnp.int32, L) + l * L for l in range(8)]

    def fire_gather(c, b):
        f = lax.rem(c, fields)
        bl = lax.div(c, fields)
        pltpu.async_copy(
            table_hbm.at[idx_v.at[f, pl.ds(bl * BLK, BLK)]], gbufs[b], gsems[b]
        )

    def wait_gather(b):
        pltpu.make_async_copy(
            table_hbm.at[idx_v.at[0, pl.ds(0, BLK)]], gbufs[b], gsems[b]
        ).wait()

    def fire_out(c, b):
        f = lax.rem(c, fields)
        bl = lax.div(c, fields)
        pltpu.async_copy(
            obufs[b], out_hbm.at[f, :, base_blk + bl, :, :], osems[b]
        )

    def wait_out(b):
        pltpu.make_async_copy(
            obufs[b], out_hbm.at[0, :, 0, :, :], osems[b]
        ).wait()

    def compute(b):
        gbuf, obuf = gbufs[b], obufs[b]

        def col_body(cc, _):
            c_hi = lax.shift_right_logical(cc, 2)
            c_lo2 = lax.mul(lax.rem(cc, 4), 2)
            for dl in range(2):
                col = jnp.full((L,), c_hi * 8 + c_lo2 + dl, jnp.int32)
                for l in range(8):
                    v = plsc.load_gather(gbuf, [row_sel[l], col])
                    obuf[c_hi, c_lo2 + dl, pl.ds(l * L, L)] = v / (1.0 + jnp.exp(-v))
            return 0

        lax.fori_loop(0, 32, col_body, 0)

    for c in range(LOOKAHEAD):
        fire_gather(c, c % NBUF)

    def group_body(g, _):
        for b in range(NBUF):
            c = g * NBUF + b
            wait_gather(b)

            @pl.when(c >= NBUF)
            def _():
                wait_out(b)

            compute(b)
            fire_out(c, b)

            @pl.when(c + LOOKAHEAD < n_chunks)
            def _():
                fire_gather(c + LOOKAHEAD, (b + LOOKAHEAD) % NBUF)

        return 0

    lax.fori_loop(0, n_chunks // NBUF, group_body, 0)

    for k in range(NBUF):
        wait_out((n_chunks - NBUF + k) % NBUF)


@jax.jit
def kernel(x, emb_weight):
    batch, fields = x.shape
    dim = emb_weight.shape[1]
    assert batch % (NW * BLK) == 0 and dim == DIM
    xt = x.T.astype(jnp.int32)

    mesh = plsc.VectorSubcoreMesh(
        core_axis_name="c", subcore_axis_name="s", num_cores=NC, num_subcores=NS
    )
    run = pl.kernel(
        _sc_body,
        out_type=jax.ShapeDtypeStruct(
            (fields, dim // 8, batch // BLK, 8, BLK), jnp.float32
        ),
        mesh=mesh,
        scratch_types=[
            pltpu.VMEM((fields, batch // NW), jnp.int32),
            [pltpu.VMEM((BLK, dim), jnp.float32) for _ in range(NBUF)],
            [pltpu.VMEM((dim // 8, 8, BLK), jnp.float32) for _ in range(NBUF)],
            [pltpu.SemaphoreType.DMA for _ in range(NBUF)],
            [pltpu.SemaphoreType.DMA for _ in range(NBUF)],
        ],
        compiler_params=pltpu.CompilerParams(
            use_tc_tiling_on_sc=False, needs_layout_passes=False
        ),
    )
    e = run(xt, emb_weight)
    return e.transpose(2, 4, 0, 1, 3).reshape(batch, fields, dim)


# speedup vs baseline: 2.0421x; 2.0421x over previous
import jax
import jax.numpy as jnp
from jax import lax
from jax.experimental import pallas as pl
from jax.experimental.pallas import tpu as pltpu
from jax.experimental.pallas import tpu_sc as plsc

NC = 2
NS = 16
L = 16
NW = NC * NS

DIM = 64
CHUNK = 128
NBUF = 4
LOOKAHEAD = 2
VPR = DIM // L


def _swish_inplace(buf):
    def row_body(r, _):
        for j in range(VPR):
            v = buf[r, pl.ds(j * L, L)]
            buf[r, pl.ds(j * L, L)] = v / (1.0 + jnp.exp(-v))
        return 0

    lax.fori_loop(0, CHUNK, row_body, 0, unroll=2)


def _sc_body(x_hbm, table_hbm, out_hbm, idx_v, bufs, gsems, osems):
    wid = lax.axis_index("s") * NC + lax.axis_index("c")
    n_chunks = x_hbm.shape[1]
    rows_per_w = n_chunks * CHUNK
    base = wid * rows_per_w

    pltpu.sync_copy(x_hbm.at[wid], idx_v)

    def fire_gather(c, b):
        pltpu.async_copy(table_hbm.at[idx_v.at[c]], bufs[b], gsems[b])

    def wait_gather(b):
        pltpu.make_async_copy(table_hbm.at[idx_v.at[0]], bufs[b], gsems[b]).wait()

    def fire_scatter(c, b):
        pltpu.async_copy(bufs[b], out_hbm.at[pl.ds(base + c * CHUNK, CHUNK)], osems[b])

    def wait_scatter(b):
        pltpu.make_async_copy(
            bufs[b], out_hbm.at[pl.ds(base, CHUNK)], osems[b]
        ).wait()

    for c in range(LOOKAHEAD):
        fire_gather(c, c % NBUF)

    def group_body(g, _):
        for b in range(NBUF):
            c = g * NBUF + b
            wait_gather(b)
            _swish_inplace(bufs[b])
            fire_scatter(c, b)

            @pl.when(c >= LOOKAHEAD)
            def _():
                wait_scatter((b - LOOKAHEAD) % NBUF)

            @pl.when(c + LOOKAHEAD < n_chunks)
            def _():
                fire_gather(c + LOOKAHEAD, (b + LOOKAHEAD) % NBUF)

        return 0

    lax.fori_loop(0, n_chunks // NBUF, group_body, 0)

    for k in range(LOOKAHEAD):
        wait_scatter((n_chunks - LOOKAHEAD + k) % NBUF)


@jax.jit
def kernel(x, emb_weight):
    batch, fields = x.shape
    dim = emb_weight.shape[1]
    n_rows = batch * fields
    assert n_rows % (NW * CHUNK * NBUF) == 0 and dim == DIM
    n_chunks = n_rows // (NW * CHUNK)

    x_split = x.reshape(NW, n_chunks, CHUNK).astype(jnp.int32)

    mesh = plsc.VectorSubcoreMesh(
        core_axis_name="c", subcore_axis_name="s", num_cores=NC, num_subcores=NS
    )
    run = pl.kernel(
        _sc_body,
        out_type=jax.ShapeDtypeStruct((n_rows, dim), jnp.float32),
        mesh=mesh,
        scratch_types=[
            pltpu.VMEM((n_chunks, CHUNK), jnp.int32),
            [pltpu.VMEM((CHUNK, dim), jnp.float32) for _ in range(NBUF)],
            [pltpu.SemaphoreType.DMA for _ in range(NBUF)],
            [pltpu.SemaphoreType.DMA for _ in range(NBUF)],
        ],
        compiler_params=pltpu.CompilerParams(use_tc_tiling_on_sc=False, needs_layout_passes=False),
    )
    out = run(x_split, emb_weight)
    return out.reshape(batch, fields, dim)


# speedup vs baseline: 2.8467x; 1.3940x over previous
import jax
import jax.numpy as jnp
from jax import lax
from jax.experimental import pallas as pl
from jax.experimental.pallas import tpu as pltpu
from jax.experimental.pallas import tpu_sc as plsc

NC = 2
NS = 16
L = 16
NW = NC * NS

DIM = 64
BLK = 128
NBUF = 4
LOOKAHEAD = 2


def _sc_body(xt_hbm, table_hbm, out_hbm, idx_v, gbufs, obufs, gsems, osems):
    wid = lax.axis_index("s") * NC + lax.axis_index("c")
    fields = xt_hbm.shape[0]
    batches = xt_hbm.shape[1]
    blocks_per_w = batches // (BLK * NW)
    n_chunks = blocks_per_w * fields
    base_blk = wid * blocks_per_w

    pltpu.sync_copy(xt_hbm.at[:, pl.ds(base_blk * BLK, blocks_per_w * BLK)], idx_v)

    lane = lax.iota(jnp.int32, L)
    chi_sel = [lax.div(lane + j * L, 8) for j in range(4)]
    clo_sel = lax.rem(lane, 8)

    def fire_gather(c, b):
        f = lax.rem(c, fields)
        bl = lax.div(c, fields)
        pltpu.async_copy(
            table_hbm.at[idx_v.at[f, pl.ds(bl * BLK, BLK)]], gbufs[b], gsems[b]
        )

    def wait_gather(b):
        pltpu.make_async_copy(
            table_hbm.at[idx_v.at[0, pl.ds(0, BLK)]], gbufs[b], gsems[b]
        ).wait()

    def fire_out(c, b):
        f = lax.rem(c, fields)
        bl = lax.div(c, fields)
        pltpu.async_copy(
            obufs[b].at[:, :, pl.ds(0, BLK)],
            out_hbm.at[f, :, base_blk + bl, :, :],
            osems[b],
        )

    def wait_out(b):
        pltpu.make_async_copy(
            obufs[b].at[:, :, pl.ds(0, BLK)], out_hbm.at[0, :, 0, :, :], osems[b]
        ).wait()

    def compute(b):
        gbuf, obuf = gbufs[b], obufs[b]

        @plsc.parallel_loop(0, BLK, unroll=2)
        def _(r):
            bvec = jnp.full((L,), r, jnp.int32)
            for j in range(4):
                v = gbuf[r, pl.ds(j * L, L)]
                s = v / (1.0 + jnp.exp(-v))
                plsc.store_scatter(obuf, [chi_sel[j], clo_sel, bvec], s)

    for c in range(LOOKAHEAD):
        fire_gather(c, c % NBUF)

    def group_body(g, _):
        for b in range(NBUF):
            c = g * NBUF + b
            wait_gather(b)

            @pl.when(c >= NBUF)
            def _():
                wait_out(b)

            compute(b)
            fire_out(c, b)

            @pl.when(c + LOOKAHEAD < n_chunks)
            def _():
                fire_gather(c + LOOKAHEAD, (b + LOOKAHEAD) % NBUF)

        return 0

    lax.fori_loop(0, n_chunks // NBUF, group_body, 0)

    for k in range(NBUF):
        wait_out((n_chunks - NBUF + k) % NBUF)


@jax.jit
def kernel(x, emb_weight):
    batch, fields = x.shape
    dim = emb_weight.shape[1]
    assert batch % (NW * BLK) == 0 and dim == DIM
    xt = x.T.astype(jnp.int32)

    mesh = plsc.VectorSubcoreMesh(
        core_axis_name="c", subcore_axis_name="s", num_cores=NC, num_subcores=NS
    )
    run = pl.kernel(
        _sc_body,
        out_type=jax.ShapeDtypeStruct(
            (fields, dim // 8, batch // BLK, 8, BLK), jnp.float32
        ),
        mesh=mesh,
        scratch_types=[
            pltpu.VMEM((fields, batch // NW), jnp.int32),
            [pltpu.VMEM((BLK, dim), jnp.float32) for _ in range(NBUF)],
            [pltpu.VMEM((dim // 8, 8, BLK + 1), jnp.float32) for _ in range(NBUF)],
            [pltpu.SemaphoreType.DMA for _ in range(NBUF)],
            [pltpu.SemaphoreType.DMA for _ in range(NBUF)],
        ],
        compiler_params=pltpu.CompilerParams(
            use_tc_tiling_on_sc=False, needs_layout_passes=False
        ),
    )
    e = run(xt, emb_weight)
    return e.transpose(2, 4, 0, 1, 3).reshape(batch, fields, dim)
